# Initial kernel scaffold; baseline (speedup 1.0000x reference)
import jax
import jax.numpy as jnp
from jax.experimental import pallas as pl


def kernel(asu_id, hkl, I, SigI, metadata, wavelength, loc, raw_scale, W1, b1, W2, b2, z_eps):
    raise NotImplementedError("write your pallas kernel here")


# speedup vs baseline: 2.3761x; 2.3761x over previous
import functools

import jax
import jax.numpy as jnp
from jax import lax
from jax.experimental import pallas as pl
from jax.experimental.pallas import tpu as pltpu
from jax.experimental.pallas import tpu_sc as plsc

B = 8
N = 8192
BN = B * N
H = 3
MC = 32
N_REFL = 100000
D_META = 5
HID = 32
N_OPS = 2
NSTREAM = N_OPS * H
DTAB = 48
ZROW = N_REFL
TROWS = N_REFL + 8

NC = 2
NS = 16
NW = NC * NS
OBS_PER_W = BN // NW
CH = 128
NCHUNK = OBS_PER_W // CH
LANES = 16

BLK = 512
GRID = BN // BLK
STEPS_PER_B = N // BLK
KLR, KLC = 100, 1000

_HALF_LOG_2PI = 0.9189385332046727


def _sc_gather_body(comps_hbm, tab_hbm, out_hbm, comp_v, idx_vs, rows_vs, sem):
    wid = lax.axis_index("s") * NC + lax.axis_index("c")
    wbase = wid * OBS_PER_W

    def chunk(ci, carry):
        base = wbase + ci * CH
        pltpu.sync_copy(comps_hbm.at[:, pl.ds(base, CH)], comp_v)

        def grp(g, c2):
            sl = pl.ds(g * LANES, LANES)
            h0 = comp_v[0, sl]
            h1 = comp_v[1, sl]
            h2 = comp_v[2, sl]
            a337 = comp_v[3, sl] * 337
            bh = h0 * 911 + h1 * 1237 + h2 * 2003
            for op in range(N_OPS):
                sb = bh if op == 0 else -bh
                for h in range(H):
                    hh = jnp.abs(sb * (h + 1) + a337)
                    rid = hh % N_REFL
                    if h > 0:
                        rid = jnp.where(hh % 3 == 0, rid, ZROW)
                    idx_vs[op * H + h][sl] = rid
            return c2

        lax.fori_loop(0, CH // LANES, grp, 0)

        for j in range(NSTREAM):
            pltpu.async_copy(tab_hbm.at[idx_vs[j]], rows_vs[j], sem).wait()
            pltpu.sync_copy(rows_vs[j], out_hbm.at[j, pl.ds(base, CH)])
        return carry

    lax.fori_loop(0, NCHUNK, chunk, 0)


@functools.cache
def _get_sc_gather():
    return pl.kernel(
        _sc_gather_body,
        out_type=jax.ShapeDtypeStruct((NSTREAM, BN, DTAB), jnp.float32),
        mesh=plsc.VectorSubcoreMesh(core_axis_name="c", subcore_axis_name="s"),
        scratch_types=[
            pltpu.VMEM((4, CH), jnp.int32),
            [pltpu.VMEM((CH,), jnp.int32) for _ in range(NSTREAM)],
            [pltpu.VMEM((CH, DTAB), jnp.float32) for _ in range(NSTREAM)],
            pltpu.SemaphoreType.DMA,
        ],
        compiler_params=pltpu.CompilerParams(use_tc_tiling_on_sc=False),
    )


def _softplus(x):
    return jnp.maximum(x, 0.0) + jnp.log(1.0 + jnp.exp(-jnp.abs(x)))


def _dense_body(g_ref, i_ref, sig_ref, meta_ref, wl_ref, loc_ref, raw_ref,
                w1_ref, b1_ref, w2_ref, b2_ref, acc_ref, kl_ref):
    i = pl.program_id(0)
    b = i // STEPS_PER_B

    @pl.when(i == 0)
    def _init():
        acc_ref[...] = jnp.zeros_like(acc_ref)
        locv = loc_ref[...]
        qs = _softplus(raw_ref[...]) + 1e-6
        kl = jnp.mean(-jnp.log(qs) + 0.5 * (qs * qs + locv * locv) - 0.5)
        kl_ref[...] = jnp.full((1, 1), kl, jnp.float32)

    iv = i_ref[...]
    sg = sig_ref[...]
    meta = meta_ref[...]
    wl = wl_ref[...]
    w1 = w1_ref[...]
    b1 = b1_ref[...]
    w2 = w2_ref[...]
    b2 = b2_ref[...]

    meta_h = meta @ w1[4:4 + D_META, :]
    base_h = (iv * w1[2:3, :] + sg * w1[3:4, :] + meta_h + b1[None, :])

    ipreds = []
    for op in range(N_OPS):
        ipred = jnp.zeros((BLK, MC), jnp.float32)
        for h in range(H):
            g = g_ref[op * H + h]
            eps = g[:, 0:MC]
            qm = g[:, MC:MC + 1]
            qs = _softplus(g[:, MC + 1:MC + 2]) + 1e-6
            wlm = wl * (1.0 / (h + 1))
            hdd = base_h + qm * w1[0:1, :] + qs * w1[1:2, :] + wlm * w1[10:11, :]
            hdd = jnp.maximum(hdd, 0.0)
            scale = jnp.dot(hdd, w2, preferred_element_type=jnp.float32) + b2[None, :]
            z_g = qm + qs * eps
            ipred = ipred + z_g * scale
        ipreds.append(ipred)

    w = 1.0 / (sg * sg)
    s_w = jnp.sum(w)
    s_wx = jnp.sum(w * iv)
    s_wxx = jnp.sum(w * iv * iv)

    vals = [s_w, s_wx, s_wxx]
    lls = []
    for op in range(N_OPS):
        ipred = ipreds[op]
        y = jnp.mean(ipred, axis=1, keepdims=True)
        vals.extend([jnp.sum(w * y), jnp.sum(w * iv * y), jnp.sum(w * y * y)])
        dev = (ipred - iv) / sg
        msq = jnp.mean(dev * dev, axis=1, keepdims=True)
        lls.append(jnp.sum(-0.5 * msq - jnp.log(sg) - _HALF_LOG_2PI))
    vals.extend(lls)

    iota16 = lax.broadcasted_iota(jnp.int32, (1, 16), 1)
    row = jnp.zeros((1, 16), jnp.float32)
    for k, v in enumerate(vals):
        row = row + v * (iota16 == k).astype(jnp.float32)
    onehot_b = (lax.broadcasted_iota(jnp.int32, (B, 1), 0) == b).astype(jnp.float32)
    acc_ref[...] += onehot_b * row


_DENSE_KW = dict(
    grid=(GRID,),
    in_specs=[
        pl.BlockSpec((NSTREAM, BLK, DTAB), lambda i: (0, i, 0)),
        pl.BlockSpec((BLK, 1), lambda i: (i, 0)),
        pl.BlockSpec((BLK, 1), lambda i: (i, 0)),
        pl.BlockSpec((BLK, D_META), lambda i: (i, 0)),
        pl.BlockSpec((BLK, 1), lambda i: (i, 0)),
        pl.BlockSpec((KLR, KLC), lambda i: (0, 0)),
        pl.BlockSpec((KLR, KLC), lambda i: (0, 0)),
        pl.BlockSpec((11, HID), lambda i: (0, 0)),
        pl.BlockSpec((HID,), lambda i: (0,)),
        pl.BlockSpec((HID, MC), lambda i: (0, 0)),
        pl.BlockSpec((MC,), lambda i: (0,)),
    ],
    out_specs=[
        pl.BlockSpec((B, 16), lambda i: (0, 0)),
        pl.BlockSpec((1, 1), lambda i: (0, 0)),
    ],
    out_shape=[
        jax.ShapeDtypeStruct((B, 16), jnp.float32),
        jax.ShapeDtypeStruct((1, 1), jnp.float32),
    ],
)

_dense_call = pl.pallas_call(_dense_body, **_DENSE_KW)


def kernel(asu_id, hkl, I, SigI, metadata, wavelength, loc, raw_scale,
           W1, b1, W2, b2, z_eps):
    hkl_f = hkl.reshape(BN, 3).astype(jnp.int32)
    comps = jnp.concatenate(
        [hkl_f.T, asu_id.reshape(1, BN).astype(jnp.int32)], axis=0)

    tab = jnp.concatenate(
        [z_eps.T, loc[:, None], raw_scale[:, None],
         jnp.zeros((N_REFL, DTAB - MC - 2), jnp.float32)], axis=1)
    tab = jnp.concatenate(
        [tab, jnp.zeros((TROWS - N_REFL, DTAB), jnp.float32)], axis=0)

    g = _get_sc_gather()(comps, tab)

    acc, klv = _dense_call(
        g, I.reshape(BN, 1), SigI.reshape(BN, 1), metadata.reshape(BN, D_META),
        wavelength.reshape(BN, 1), loc.reshape(KLR, KLC),
        raw_scale.reshape(KLR, KLC), W1, b1, W2, b2)

    kl = klv[0, 0]
    ll = acc[:, 9:11] / jnp.float32(N)
    op_idx = jnp.argmax(ll, axis=-1)
    elbo = -jnp.mean(jnp.max(ll, axis=-1)) + kl

    s_w = jnp.sum(acc[:, 0])
    s_wx = jnp.sum(acc[:, 1])
    s_wxx = jnp.sum(acc[:, 2])
    per_op = acc[:, 3:9].reshape(B, N_OPS, 3)
    sel = jnp.take_along_axis(per_op, op_idx[:, None, None], axis=1)[:, 0, :]
    s_wy = jnp.sum(sel[:, 0])
    s_wxy = jnp.sum(sel[:, 1])
    s_wyy = jnp.sum(sel[:, 2])
    z = 1.0 / s_w
    mx = z * s_wx
    my = z * s_wy
    cxy = z * s_wxy - mx * my
    cx = z * s_wxx - mx * mx
    cy = z * s_wyy - my * my
    cc = cxy / jnp.sqrt(cx * cy)
    return elbo, kl, cc, op_idx
